# baseline (speedup 1.0000x reference)
import jax
import jax.numpy as jnp
from jax import lax
from jax.experimental import pallas as pl
from jax.experimental.pallas import tpu as pltpu
from jax.experimental.pallas import tpu_sc as plsc


def _scs_body(x_hbm, out_hbm, buf, res):
    cid = lax.axis_index("c")

    @pl.when(cid == 0)
    def _():
        pltpu.sync_copy(x_hbm.at[pl.ds(1, 2)], buf)
        a = buf[0, 0]
        b = buf[1, 0]
        neq = jnp.logical_or(a != a, b != b)
        res[0] = jnp.where(neq, 1.0, 0.0).astype(jnp.float32)
        pltpu.sync_copy(res, out_hbm)


@jax.jit
def _sc_gather_compare(x):
    mesh = plsc.ScalarSubcoreMesh(axis_name="c", num_cores=1)
    out = pl.kernel(
        _scs_body,
        out_type=jax.ShapeDtypeStruct((1,), jnp.float32),
        mesh=mesh,
        scratch_types=[
            pltpu.SMEM((2, 64), jnp.float32),
            pltpu.SMEM((1,), jnp.float32),
        ],
    )(x)
    return out[0]


def kernel(x):
    return _sc_gather_compare(x)
